# Initial kernel scaffold; baseline (speedup 1.0000x reference)
import jax
import jax.numpy as jnp
from jax.experimental import pallas as pl


def kernel(expression, batch_labels):
    raise NotImplementedError("write your pallas kernel here")


# speedup vs baseline: 6.4571x; 6.4571x over previous
import functools

import jax
import jax.numpy as jnp
from jax import lax
from jax.experimental import pallas as pl

NB = 8
N = 100000
G = 512
BLK = 2000
GRID = N // BLK


def _stats_body(x_ref, lab_ref, seg_ref, ssq_ref, cnt_ref):
    i = pl.program_id(0)

    @pl.when(i == 0)
    def _():
        seg_ref[...] = jnp.zeros_like(seg_ref)
        ssq_ref[...] = jnp.zeros_like(ssq_ref)
        cnt_ref[...] = jnp.zeros_like(cnt_ref)

    x = x_ref[...]
    labels = lab_ref[0, 0, :]
    onehot = (labels[:, None] == lax.broadcasted_iota(jnp.int32, (1, NB), 1)
              ).astype(jnp.float32)
    seg_ref[...] += lax.dot_general(
        onehot, x, (((0,), (0,)), ((), ())),
        preferred_element_type=jnp.float32)
    ssq_ref[...] += jnp.sum(x * x, axis=0, keepdims=True)
    cnt_ref[...] += jnp.sum(onehot, axis=0, keepdims=True)


def _apply_body(x_ref, lab_ref, seg_ref, ssq_ref, cnt_ref, out_ref):
    seg = seg_ref[...]
    ssq = ssq_ref[...]
    cnt = cnt_ref[...]
    gm = jnp.sum(seg, axis=0, keepdims=True) / N
    gv = ssq / N - gm * gm
    inv_std = 1.0 / (jnp.sqrt(gv) + 1e-8)
    recip = 1.0 / jnp.maximum(cnt, 1.0)

    labels = lab_ref[0, 0, :]
    onehot = (labels[:, None] == lax.broadcasted_iota(jnp.int32, (1, NB), 1)
              ).astype(jnp.float32)
    bmean = lax.dot_general(
        onehot * recip, seg, (((1,), (0,)), ((), ())),
        preferred_element_type=jnp.float32)
    out_ref[...] = x_ref[...] - (bmean - gm) * inv_std


@jax.jit
def kernel(expression, batch_labels):
    labels3 = batch_labels.reshape(GRID, 1, BLK)

    seg, ssq, cnt = pl.pallas_call(
        _stats_body,
        grid=(GRID,),
        in_specs=[
            pl.BlockSpec((BLK, G), lambda i: (i, 0)),
            pl.BlockSpec((1, 1, BLK), lambda i: (i, 0, 0)),
        ],
        out_specs=[
            pl.BlockSpec((NB, G), lambda i: (0, 0)),
            pl.BlockSpec((1, G), lambda i: (0, 0)),
            pl.BlockSpec((1, NB), lambda i: (0, 0)),
        ],
        out_shape=[
            jax.ShapeDtypeStruct((NB, G), jnp.float32),
            jax.ShapeDtypeStruct((1, G), jnp.float32),
            jax.ShapeDtypeStruct((1, NB), jnp.float32),
        ],
    )(expression, labels3)

    corrected = pl.pallas_call(
        _apply_body,
        grid=(GRID,),
        in_specs=[
            pl.BlockSpec((BLK, G), lambda i: (i, 0)),
            pl.BlockSpec((1, 1, BLK), lambda i: (i, 0, 0)),
            pl.BlockSpec((NB, G), lambda i: (0, 0)),
            pl.BlockSpec((1, G), lambda i: (0, 0)),
            pl.BlockSpec((1, NB), lambda i: (0, 0)),
        ],
        out_specs=pl.BlockSpec((BLK, G), lambda i: (i, 0)),
        out_shape=jax.ShapeDtypeStruct((N, G), jnp.float32),
    )(expression, labels3, seg, ssq, cnt)

    return corrected
